# Initial kernel scaffold; baseline (speedup 1.0000x reference)
import jax
import jax.numpy as jnp
from jax.experimental import pallas as pl


def kernel(x, edge_index, edge_attr, W_lin, b_lin, W_lin2, b_lin2, W1, b1, W2, b2):
    raise NotImplementedError("write your pallas kernel here")


# speedup vs baseline: 4.6953x; 4.6953x over previous
import jax
import jax.numpy as jnp
from jax import lax
from jax.experimental import pallas as pl
from jax.experimental.pallas import tpu as pltpu
from jax.experimental.pallas import tpu_sc as plsc

N = 10000
E = 320000
D = 128
DE = 16
HALF = E // 2
NC = 2
NS = 16
CH = 128
CHUNKS_PER_SC = HALF // CH
BASE_CHUNKS = CHUNKS_PER_SC // NS
EXTRA = CHUNKS_PER_SC - BASE_CHUNKS * NS
MAXCH = BASE_CHUNKS + 1
ROWS_PER_TILE = N // NS
BN = 2000
BE = 2000

_f32 = jnp.float32


def _tables_body(x_ref, wd_ref, ws_ref, bd_ref, td_ref, ts_ref):
    xb = x_ref[...]
    td_ref[0] = jnp.dot(xb, wd_ref[0], preferred_element_type=_f32) + bd_ref[0]
    ts_ref[0] = jnp.dot(xb, ws_ref[0], preferred_element_type=_f32)


_tables_call = pl.pallas_call(
    _tables_body,
    grid=(2, N // BN),
    in_specs=[
        pl.BlockSpec((BN, D), lambda i, j: (j, 0)),
        pl.BlockSpec((1, D, D), lambda i, j: (i, 0, 0)),
        pl.BlockSpec((1, D, D), lambda i, j: (i, 0, 0)),
        pl.BlockSpec((1, 1, D), lambda i, j: (i, 0, 0)),
    ],
    out_specs=[
        pl.BlockSpec((1, BN, D), lambda i, j: (i, j, 0)),
        pl.BlockSpec((1, BN, D), lambda i, j: (i, j, 0)),
    ],
    out_shape=[jax.ShapeDtypeStruct((2, N, D), _f32)] * 2,
)


def _ea_body(ea_ref, we_ref, eo_ref):
    eo_ref[0] = jnp.dot(ea_ref[0], we_ref[0], preferred_element_type=_f32)


_ea_call = pl.pallas_call(
    _ea_body,
    grid=(2, HALF // BE),
    in_specs=[
        pl.BlockSpec((1, BE, DE), lambda i, j: (i, j, 0)),
        pl.BlockSpec((1, DE, D), lambda i, j: (i, 0, 0)),
    ],
    out_specs=pl.BlockSpec((1, BE, D), lambda i, j: (i, j, 0)),
    out_shape=jax.ShapeDtypeStruct((2, HALF, D), _f32),
)


def _final_body(agg_ref, x_ref, w1_ref, b1_ref, w2_ref, b2_ref, o_ref):
    a = agg_ref[0] + agg_ref[1] + x_ref[...]
    h = jnp.maximum(jnp.dot(a, w1_ref[...], preferred_element_type=_f32) + b1_ref[...], 0.0)
    o_ref[...] = jnp.dot(h, w2_ref[...], preferred_element_type=_f32) + b2_ref[...]


_final_call = pl.pallas_call(
    _final_body,
    grid=(N // BN,),
    in_specs=[
        pl.BlockSpec((2, BN, D), lambda j: (0, j, 0)),
        pl.BlockSpec((BN, D), lambda j: (j, 0)),
        pl.BlockSpec((D, D), lambda j: (0, 0)),
        pl.BlockSpec((1, D), lambda j: (0, 0)),
        pl.BlockSpec((D, D), lambda j: (0, 0)),
        pl.BlockSpec((1, D), lambda j: (0, 0)),
    ],
    out_specs=pl.BlockSpec((BN, D), lambda j: (j, 0)),
    out_shape=jax.ShapeDtypeStruct((N, D), _f32),
)


def _sc_edge_body(tdst_hbm, tsrc_hbm, ea_hbm, src_hbm, dst_hbm, agg_hbm,
                  idx_d, idx_s, idx_g, buf_a, buf_c, buf_e, agg_sh, sem):
    c = lax.axis_index("c")
    s = lax.axis_index("s")
    c_n = c * N
    nch = BASE_CHUNKS

    zero16 = jnp.zeros((16,), _f32)

    @pl.loop(0, CH)
    def _(i):
        for j in range(8):
            buf_a[i, pl.ds(j * 16, 16)] = zero16

    row0 = s * ROWS_PER_TILE
    for k in range(ROWS_PER_TILE // CH):
        pltpu.sync_copy(buf_a, agg_sh.at[pl.ds(row0 + k * CH, CH)])
    _rem = ROWS_PER_TILE % CH
    pltpu.sync_copy(buf_a.at[pl.ds(0, _rem)],
                    agg_sh.at[pl.ds(row0 + (ROWS_PER_TILE // CH) * CH, _rem)])

    base_chunk = s * BASE_CHUNKS + jnp.minimum(s, EXTRA)
    chunk_lo = c * CHUNKS_PER_SC + base_chunk

    plsc.subcore_barrier()

    def do_chunk(g):
        e0 = c * HALF + (base_chunk + g) * CH
        ci1 = pltpu.async_copy(dst_hbm.at[pl.ds(chunk_lo + g, 1)], idx_d, sem)
        ci2 = pltpu.async_copy(src_hbm.at[pl.ds(chunk_lo + g, 1)], idx_s, sem)
        ci1.wait()
        ci2.wait()
        for j in range(8):
            sl = pl.ds(j * 16, 16)
            idx_g[0, 0, sl] = idx_d[0, 0, sl] + c_n
            idx_s[0, 0, sl] = idx_s[0, 0, sl] + c_n
        cp1 = pltpu.async_copy(tdst_hbm.at[idx_g.at[0, 0]], buf_a, sem)
        cp2 = pltpu.async_copy(tsrc_hbm.at[idx_s.at[0, 0]], buf_c, sem)
        cp3 = pltpu.async_copy(ea_hbm.at[pl.ds(e0, CH)], buf_e, sem)
        cp1.wait()
        cp2.wait()
        cp3.wait()

        @pl.loop(0, CH)
        def _(i):
            for j in range(8):
                sl = pl.ds(j * 16, 16)
                v = buf_a[i, sl] + buf_c[i, sl] + buf_e[i, sl]
                buf_a[i, sl] = jnp.maximum(v, 0.0)

        pltpu.sync_copy(buf_a, agg_sh.at[idx_d.at[0, 0]], add=True)

    @pl.loop(0, BASE_CHUNKS)
    def _(g):
        do_chunk(g)

    @pl.when(s < EXTRA)
    def _():
        do_chunk(nch)

    plsc.subcore_barrier()

    @pl.when(s < 10)
    def _():
        pltpu.sync_copy(agg_sh.at[pl.ds(s * 1000, 1000)],
                        agg_hbm.at[pl.ds(c * N + s * 1000, 1000)])


_sc_mesh = plsc.VectorSubcoreMesh(core_axis_name="c", subcore_axis_name="s")

_sc_edge_call = pl.kernel(
    _sc_edge_body,
    out_type=jax.ShapeDtypeStruct((2 * N, D), _f32),
    mesh=_sc_mesh,
    scratch_types=[
        pltpu.VMEM((1, 1, CH), jnp.int32),
        pltpu.VMEM((1, 1, CH), jnp.int32),
        pltpu.VMEM((1, 1, CH), jnp.int32),
        pltpu.VMEM((CH, D), _f32),
        pltpu.VMEM((CH, D), _f32),
        pltpu.VMEM((CH, D), _f32),
        pltpu.VMEM_SHARED((N, D), _f32),
        pltpu.SemaphoreType.DMA,
    ],
)


def kernel(x, edge_index, edge_attr, W_lin, b_lin, W_lin2, b_lin2, W1, b1, W2, b2):
    src = edge_index[0].reshape(CHUNKS_PER_SC * 2, 1, CH)
    dst = edge_index[1].reshape(CHUNKS_PER_SC * 2, 1, CH)

    w_dst = jnp.stack([W_lin[:D], W_lin2[D + DE:]])
    w_src = jnp.stack([W_lin[D + DE:], W_lin2[:D]])
    w_ea = jnp.stack([W_lin[D:D + DE], W_lin2[D:D + DE]])
    b_msg = jnp.stack([b_lin, b_lin2])[:, None, :]

    tdst, tsrc = _tables_call(x, w_dst, w_src, b_msg)
    ea_rows = _ea_call(edge_attr.reshape(2, HALF, DE), w_ea)

    agg = _sc_edge_call(
        tdst.reshape(2 * N, D),
        tsrc.reshape(2 * N, D),
        ea_rows.reshape(E, D),
        src,
        dst,
    )

    return _final_call(agg.reshape(2, N, D), x, W1, b1.reshape(1, D),
                       W2, b2.reshape(1, D))


# speedup vs baseline: 6.5341x; 1.3916x over previous
import jax
import jax.numpy as jnp
from jax import lax
from jax.experimental import pallas as pl
from jax.experimental.pallas import tpu as pltpu
from jax.experimental.pallas import tpu_sc as plsc

N = 10000
E = 320000
D = 128
DE = 16
HALF = E // 2
NC = 2
NS = 16
CH = 64
CHUNKS_PER_SC = HALF // CH
BASE_CHUNKS = CHUNKS_PER_SC // NS
EXTRA = CHUNKS_PER_SC - BASE_CHUNKS * NS
ROWS_PER_TILE = N // NS
BN = 2000
BE = 2000

_f32 = jnp.float32


def _tables_body(x_ref, wd_ref, ws_ref, bd_ref, td_ref, ts_ref):
    xb = x_ref[...]
    td_ref[0] = jnp.dot(xb, wd_ref[0], preferred_element_type=_f32) + bd_ref[0]
    ts_ref[0] = jnp.dot(xb, ws_ref[0], preferred_element_type=_f32)


_tables_call = pl.pallas_call(
    _tables_body,
    grid=(2, N // BN),
    in_specs=[
        pl.BlockSpec((BN, D), lambda i, j: (j, 0)),
        pl.BlockSpec((1, D, D), lambda i, j: (i, 0, 0)),
        pl.BlockSpec((1, D, D), lambda i, j: (i, 0, 0)),
        pl.BlockSpec((1, 1, D), lambda i, j: (i, 0, 0)),
    ],
    out_specs=[
        pl.BlockSpec((1, BN, D), lambda i, j: (i, j, 0)),
        pl.BlockSpec((1, BN, D), lambda i, j: (i, j, 0)),
    ],
    out_shape=[jax.ShapeDtypeStruct((2, N, D), _f32)] * 2,
)


def _ea_body(ea_ref, we_ref, eo_ref):
    eo_ref[0] = jnp.dot(ea_ref[0], we_ref[0], preferred_element_type=_f32)


_ea_call = pl.pallas_call(
    _ea_body,
    grid=(2, HALF // BE),
    in_specs=[
        pl.BlockSpec((1, BE, DE), lambda i, j: (i, j, 0)),
        pl.BlockSpec((1, DE, D), lambda i, j: (i, 0, 0)),
    ],
    out_specs=pl.BlockSpec((1, BE, D), lambda i, j: (i, j, 0)),
    out_shape=jax.ShapeDtypeStruct((2, HALF, D), _f32),
)


def _final_body(agg_ref, x_ref, w1_ref, b1_ref, w2_ref, b2_ref, o_ref):
    a = agg_ref[0] + agg_ref[1] + x_ref[...]
    h = jnp.maximum(jnp.dot(a, w1_ref[...], preferred_element_type=_f32) + b1_ref[...], 0.0)
    o_ref[...] = jnp.dot(h, w2_ref[...], preferred_element_type=_f32) + b2_ref[...]


_final_call = pl.pallas_call(
    _final_body,
    grid=(N // BN,),
    in_specs=[
        pl.BlockSpec((2, BN, D), lambda j: (0, j, 0)),
        pl.BlockSpec((BN, D), lambda j: (j, 0)),
        pl.BlockSpec((D, D), lambda j: (0, 0)),
        pl.BlockSpec((1, D), lambda j: (0, 0)),
        pl.BlockSpec((D, D), lambda j: (0, 0)),
        pl.BlockSpec((1, D), lambda j: (0, 0)),
    ],
    out_specs=pl.BlockSpec((BN, D), lambda j: (j, 0)),
    out_shape=jax.ShapeDtypeStruct((N, D), _f32),
)


def _sc_edge_body(tdst_hbm, tsrc_hbm, ea_hbm, src_hbm, dst_hbm, agg_hbm,
                  idxd0, idxr0, idxs0, idxg0, idxc0, a0, c0, e0,
                  idxd1, idxr1, idxs1, idxg1, idxc1, a1, c1, e1,
                  agg_sh, semi0, semi1, semg0, semg1):
    c = lax.axis_index("c")
    s = lax.axis_index("s")
    c_n = c * N

    idxd = (idxd0, idxd1)
    idxr = (idxr0, idxr1)
    idxs = (idxs0, idxs1)
    idxg = (idxg0, idxg1)
    idxc = (idxc0, idxc1)
    buf_a = (a0, a1)
    buf_c = (c0, c1)
    buf_e = (e0, e1)
    semi = (semi0, semi1)
    semg = (semg0, semg1)

    zero16 = jnp.zeros((16,), _f32)

    @pl.loop(0, CH)
    def _(i):
        for j in range(8):
            a0[i, pl.ds(j * 16, 16)] = zero16

    row0 = s * ROWS_PER_TILE
    for k in range(ROWS_PER_TILE // CH):
        pltpu.sync_copy(a0, agg_sh.at[pl.ds(row0 + k * CH, CH)])
    _rem = ROWS_PER_TILE % CH
    pltpu.sync_copy(a0.at[pl.ds(0, _rem)],
                    agg_sh.at[pl.ds(row0 + (ROWS_PER_TILE // CH) * CH, _rem)])

    base_chunk = s * BASE_CHUNKS + jnp.minimum(s, EXTRA)
    chunk_lo = c * CHUNKS_PER_SC + base_chunk

    plsc.subcore_barrier()


    def issue_idx(row, b):
        pltpu.async_copy(dst_hbm.at[pl.ds(row, 1)], idxd[b], semi[b])
        pltpu.async_copy(src_hbm.at[pl.ds(row, 1)], idxr[b], semi[b])

    def prep_gather(row, b):
        pltpu.make_async_copy(dst_hbm.at[pl.ds(0, 1)], idxd[b], semi[b]).wait()
        pltpu.make_async_copy(src_hbm.at[pl.ds(0, 1)], idxr[b], semi[b]).wait()
        for j in range(CH // 16):
            sl = pl.ds(j * 16, 16)
            idxc[b][0, 0, sl] = idxd[b][0, 0, sl]
            idxg[b][0, 0, sl] = idxd[b][0, 0, sl] + c_n
            idxs[b][0, 0, sl] = idxr[b][0, 0, sl] + c_n
        pltpu.async_copy(tdst_hbm.at[idxg[b].at[0, 0]], buf_a[b], semg[b])
        pltpu.async_copy(tsrc_hbm.at[idxs[b].at[0, 0]], buf_c[b], semg[b])
        pltpu.async_copy(ea_hbm.at[pl.ds(row * CH, CH)], buf_e[b], semg[b])

    def process(b):
        pltpu.make_async_copy(tdst_hbm.at[idxg[b].at[0, 0]], buf_a[b],
                              semg[b]).wait()
        pltpu.make_async_copy(tsrc_hbm.at[idxs[b].at[0, 0]], buf_c[b],
                              semg[b]).wait()
        pltpu.make_async_copy(ea_hbm.at[pl.ds(0, CH)], buf_e[b],
                              semg[b]).wait()

        @pl.loop(0, CH)
        def _(i):
            for j in range(8):
                sl = pl.ds(j * 16, 16)
                v = buf_a[b][i, sl] + buf_c[b][i, sl] + buf_e[b][i, sl]
                buf_a[b][i, sl] = jnp.maximum(v, 0.0)

        pltpu.sync_copy(buf_a[b], agg_sh.at[idxc[b].at[0, 0]], add=True)

    issue_idx(chunk_lo, 0)
    issue_idx(chunk_lo + 1, 1)
    prep_gather(chunk_lo, 0)

    @pl.loop(0, BASE_CHUNKS - 2, step=2)
    def _(g):
        row = chunk_lo + g
        prep_gather(row + 1, 1)
        issue_idx(row + 2, 0)
        process(0)
        prep_gather(row + 2, 0)
        issue_idx(row + 3, 1)
        process(1)

    prep_gather(chunk_lo + BASE_CHUNKS - 1, 1)
    process(0)
    process(1)

    @pl.when(s < EXTRA)
    def _():
        r = chunk_lo + BASE_CHUNKS
        issue_idx(r, 0)
        prep_gather(r, 0)
        process(0)

    plsc.subcore_barrier()

    @pl.when(s < 10)
    def _():
        pltpu.sync_copy(agg_sh.at[pl.ds(s * 1000, 1000)],
                        agg_hbm.at[pl.ds(c * N + s * 1000, 1000)])


_sc_mesh = plsc.VectorSubcoreMesh(core_axis_name="c", subcore_axis_name="s")

_sc_scratch_set = [
    pltpu.VMEM((1, 1, CH), jnp.int32),
    pltpu.VMEM((1, 1, CH), jnp.int32),
    pltpu.VMEM((1, 1, CH), jnp.int32),
    pltpu.VMEM((1, 1, CH), jnp.int32),
    pltpu.VMEM((1, 1, CH), jnp.int32),
    pltpu.VMEM((CH, D), _f32),
    pltpu.VMEM((CH, D), _f32),
    pltpu.VMEM((CH, D), _f32),
]

_sc_edge_call = pl.kernel(
    _sc_edge_body,
    out_type=jax.ShapeDtypeStruct((2 * N, D), _f32),
    mesh=_sc_mesh,
    scratch_types=_sc_scratch_set + _sc_scratch_set + [
        pltpu.VMEM_SHARED((N, D), _f32),
        pltpu.SemaphoreType.DMA,
        pltpu.SemaphoreType.DMA,
        pltpu.SemaphoreType.DMA,
        pltpu.SemaphoreType.DMA,
    ],
)


def kernel(x, edge_index, edge_attr, W_lin, b_lin, W_lin2, b_lin2, W1, b1, W2, b2):
    src = edge_index[0].reshape(CHUNKS_PER_SC * 2, 1, CH)
    dst = edge_index[1].reshape(CHUNKS_PER_SC * 2, 1, CH)

    w_dst = jnp.stack([W_lin[:D], W_lin2[D + DE:]])
    w_src = jnp.stack([W_lin[D + DE:], W_lin2[:D]])
    w_ea = jnp.stack([W_lin[D:D + DE], W_lin2[D:D + DE]])
    b_msg = jnp.stack([b_lin, b_lin2])[:, None, :]

    tdst, tsrc = _tables_call(x, w_dst, w_src, b_msg)
    ea_rows = _ea_call(edge_attr.reshape(2, HALF, DE), w_ea)

    agg = _sc_edge_call(
        tdst.reshape(2 * N, D),
        tsrc.reshape(2 * N, D),
        ea_rows.reshape(E, D),
        src,
        dst,
    )

    return _final_call(agg.reshape(2, N, D), x, W1, b1.reshape(1, D),
                       W2, b2.reshape(1, D))
